# baseline (speedup 1.0000x reference)
import jax
import jax.numpy as jnp
from jax.experimental import pallas as pl

_N, _D = 100000, 256
_BR = 2048


def _body(x_ref, a_ref, h_ref, w_ref, o_ref):
    beta = jnp.clip(1.0 - w_ref[...] * a_ref[...], 0.0, 1.0)
    beta = beta.reshape(_BR, 1)
    o_ref[...] = h_ref[...] * beta + x_ref[...]


def kernel(x, agg_n, his_x, inv_w):
    return pl.pallas_call(
        _body,
        grid=((_N + _BR - 1) // _BR,),
        in_specs=[
            pl.BlockSpec((_BR, _D), lambda i: (i, 0)),
            pl.BlockSpec((_BR,), lambda i: (i,)),
            pl.BlockSpec((_BR, _D), lambda i: (i, 0)),
            pl.BlockSpec((_BR,), lambda i: (i,)),
        ],
        out_specs=pl.BlockSpec((_BR, _D), lambda i: (i, 0)),
        out_shape=jax.ShapeDtypeStruct((_N, _D), jnp.float32),
    )(x, agg_n, his_x, inv_w)


# speedup vs baseline: 1.0482x; 1.0482x over previous
import jax
import jax.numpy as jnp
from jax.experimental import pallas as pl

_N, _D = 100000, 256
_BR = 4096


def _body(x_ref, a_ref, h_ref, w_ref, o_ref):
    beta = jnp.clip(1.0 - w_ref[...] * a_ref[...], 0.0, 1.0)
    beta = beta.reshape(_BR, 1)
    o_ref[...] = h_ref[...] * beta + x_ref[...]


def kernel(x, agg_n, his_x, inv_w):
    return pl.pallas_call(
        _body,
        grid=((_N + _BR - 1) // _BR,),
        in_specs=[
            pl.BlockSpec((_BR, _D), lambda i: (i, 0)),
            pl.BlockSpec((_BR,), lambda i: (i,)),
            pl.BlockSpec((_BR, _D), lambda i: (i, 0)),
            pl.BlockSpec((_BR,), lambda i: (i,)),
        ],
        out_specs=pl.BlockSpec((_BR, _D), lambda i: (i, 0)),
        out_shape=jax.ShapeDtypeStruct((_N, _D), jnp.float32),
    )(x, agg_n, his_x, inv_w)
